# baseline (speedup 1.0000x reference)
import functools

import jax
import jax.numpy as jnp
from jax import lax
from jax.experimental import pallas as pl
from jax.experimental.pallas import tpu as pltpu
from jax.experimental.pallas import tpu_sc as plsc

_EMBED_DIM = 768
_NGRIDS = 256
_LANES = 16

_NC = 2
_NS = 16
_NW = _NC * _NS

_KSPLIT = 8
_CHUNK = 16
_NBUF = 4
_ROWBLK = 512


def _div224(v):
    return ((v >> 5) * 9363) >> 16


def _gather_body(blk_tokens, c0_hbm, c1_hbm, table_hbm, out_hbm,
                 c0_v, c1_v, idx_v, gb, *sems):
    rows_per_w = blk_tokens // _NW
    n_chunks = rows_per_w // _CHUNK
    gsem = sems[0:_NBUF]
    osem = sems[_NBUF:2 * _NBUF]
    wid = lax.axis_index("s") * _NC + lax.axis_index("c")
    base = wid * rows_per_w

    pltpu.sync_copy(c0_hbm.at[pl.ds(base, rows_per_w)], c0_v)
    pltpu.sync_copy(c1_hbm.at[pl.ds(base, rows_per_w)], c1_v)

    def idx_body(i, _):
        ci = i // (_CHUNK // _LANES)
        off = (i % (_CHUNK // _LANES)) * _LANES
        a = c0_v[pl.ds(i * _LANES, _LANES)]
        b = c1_v[pl.ds(i * _LANES, _LANES)]
        idx_v[ci, pl.ds(off, _LANES)] = _div224(a) * _NGRIDS + _div224(b)
        return 0

    lax.fori_loop(0, rows_per_w // _LANES, idx_body, 0)

    def out_slice(ci):
        return out_hbm.at[pl.ds(base + ci * _CHUNK, _CHUNK)]

    def stage_in(i, b):
        @pl.when(i >= _NBUF)
        def _():
            pltpu.make_async_copy(gb.at[b], out_slice(i), osem[b]).wait()
        pltpu.async_copy(table_hbm.at[idx_v.at[i]], gb.at[b], gsem[b])

    def stage_out(cj, bj):
        pltpu.make_async_copy(table_hbm.at[idx_v.at[cj]], gb.at[bj],
                              gsem[bj]).wait()
        pltpu.async_copy(gb.at[bj], out_slice(cj), osem[bj])

    def group_body(g, _):
        for b in range(_NBUF):
            i = g * _NBUF + b
            stage_in(i, b)

            @pl.when(i >= 1)
            def _():
                stage_out(i - 1, (b - 1) % _NBUF)
        return 0

    lax.fori_loop(0, n_chunks // _NBUF, group_body, 0)

    last = n_chunks - 1
    stage_out(last, last % _NBUF)
    for k in range(_NBUF):
        ci = n_chunks - _NBUF + k
        pltpu.make_async_copy(gb.at[ci % _NBUF], out_slice(ci),
                              osem[ci % _NBUF]).wait()


def _make_gather(blk_tokens, d):
    mesh = plsc.VectorSubcoreMesh(core_axis_name="c", subcore_axis_name="s")
    rows_per_w = blk_tokens // _NW
    return pl.kernel(
        functools.partial(_gather_body, blk_tokens),
        out_type=jax.ShapeDtypeStruct((blk_tokens, d), jnp.float32),
        mesh=mesh,
        scratch_types=[
            pltpu.VMEM((rows_per_w,), jnp.int32),
            pltpu.VMEM((rows_per_w,), jnp.int32),
            pltpu.VMEM((rows_per_w // _CHUNK, _CHUNK), jnp.int32),
            pltpu.VMEM((_NBUF, _CHUNK, d), jnp.float32),
        ] + [pltpu.SemaphoreType.DMA] * (2 * _NBUF),
    )


def _add_block_body(buf_ref, g_ref, out_ref):
    out_ref[...] = buf_ref[...] + g_ref[...]


def _add_block(buf, g, k):
    n_tokens, d = buf.shape
    blk_tokens = g.shape[0]
    nblk = blk_tokens // _ROWBLK
    row0 = k * (blk_tokens // _ROWBLK)
    return pl.pallas_call(
        _add_block_body,
        grid=(nblk,),
        in_specs=[
            pl.BlockSpec((_ROWBLK, d), lambda i, r0=row0: (r0 + i, 0)),
            pl.BlockSpec((_ROWBLK, d), lambda i: (i, 0)),
        ],
        out_specs=pl.BlockSpec((_ROWBLK, d), lambda i, r0=row0: (r0 + i, 0)),
        out_shape=jax.ShapeDtypeStruct((n_tokens, d), jnp.float32),
        input_output_aliases={0: 0},
    )(buf, g)


@jax.jit
def kernel(x, coords, pos_embed):
    b, n, d = x.shape
    n_tokens = b * n
    x2d = x.reshape(n_tokens, d)
    ci32 = coords.astype(jnp.int32)
    c0 = ci32[..., 0].reshape(n_tokens)
    c1 = ci32[..., 1].reshape(n_tokens)
    table = pos_embed[0]

    blk = n_tokens // _KSPLIT
    gather = _make_gather(blk, d)
    gs = [
        gather(c0[k * blk:(k + 1) * blk], c1[k * blk:(k + 1) * blk], table)
        for k in range(_KSPLIT)
    ]
    buf = x2d
    for k in range(_KSPLIT):
        buf = _add_block(buf, gs[k], k)
    return buf.reshape(b, n, d)


# speedup vs baseline: 1.0346x; 1.0346x over previous
import functools

import jax
import jax.numpy as jnp
from jax import lax
from jax.experimental import pallas as pl
from jax.experimental.pallas import tpu as pltpu
from jax.experimental.pallas import tpu_sc as plsc

_EMBED_DIM = 768
_NGRIDS = 256
_LANES = 16

_NC = 2
_NS = 16
_NW = _NC * _NS

_KSPLIT = 4
_CHUNK = 32
_NBUF = 4
_ROWBLK = 512


def _div224(v):
    return ((v >> 5) * 9363) >> 16


def _gather_body(blk_tokens, c0_hbm, c1_hbm, table_hbm, out_hbm,
                 c0_v, c1_v, idx_v, gb, *sems):
    rows_per_w = blk_tokens // _NW
    n_chunks = rows_per_w // _CHUNK
    gsem = sems[0:_NBUF]
    osem = sems[_NBUF:2 * _NBUF]
    wid = lax.axis_index("s") * _NC + lax.axis_index("c")
    base = wid * rows_per_w

    pltpu.sync_copy(c0_hbm.at[pl.ds(base, rows_per_w)], c0_v)
    pltpu.sync_copy(c1_hbm.at[pl.ds(base, rows_per_w)], c1_v)

    def idx_body(i, _):
        ci = i // (_CHUNK // _LANES)
        off = (i % (_CHUNK // _LANES)) * _LANES
        a = c0_v[pl.ds(i * _LANES, _LANES)]
        b = c1_v[pl.ds(i * _LANES, _LANES)]
        idx_v[ci, pl.ds(off, _LANES)] = _div224(a) * _NGRIDS + _div224(b)
        return 0

    lax.fori_loop(0, rows_per_w // _LANES, idx_body, 0)

    def out_slice(ci):
        return out_hbm.at[pl.ds(base + ci * _CHUNK, _CHUNK)]

    def stage_in(i, b):
        @pl.when(i >= _NBUF)
        def _():
            pltpu.make_async_copy(gb.at[b], out_slice(i), osem[b]).wait()
        pltpu.async_copy(table_hbm.at[idx_v.at[i]], gb.at[b], gsem[b])

    def stage_out(cj, bj):
        pltpu.make_async_copy(table_hbm.at[idx_v.at[cj]], gb.at[bj],
                              gsem[bj]).wait()
        pltpu.async_copy(gb.at[bj], out_slice(cj), osem[bj])

    def group_body(g, _):
        for b in range(_NBUF):
            i = g * _NBUF + b
            stage_in(i, b)

            @pl.when(i >= 1)
            def _():
                stage_out(i - 1, (b - 1) % _NBUF)
        return 0

    lax.fori_loop(0, n_chunks // _NBUF, group_body, 0)

    last = n_chunks - 1
    stage_out(last, last % _NBUF)
    for k in range(_NBUF):
        ci = n_chunks - _NBUF + k
        pltpu.make_async_copy(gb.at[ci % _NBUF], out_slice(ci),
                              osem[ci % _NBUF]).wait()


def _make_gather(blk_tokens, d):
    mesh = plsc.VectorSubcoreMesh(core_axis_name="c", subcore_axis_name="s")
    rows_per_w = blk_tokens // _NW
    return pl.kernel(
        functools.partial(_gather_body, blk_tokens),
        out_type=jax.ShapeDtypeStruct((blk_tokens, d), jnp.float32),
        mesh=mesh,
        scratch_types=[
            pltpu.VMEM((rows_per_w,), jnp.int32),
            pltpu.VMEM((rows_per_w,), jnp.int32),
            pltpu.VMEM((rows_per_w // _CHUNK, _CHUNK), jnp.int32),
            pltpu.VMEM((_NBUF, _CHUNK, d), jnp.float32),
        ] + [pltpu.SemaphoreType.DMA] * (2 * _NBUF),
    )


def _add_first_body(x_ref, g_ref, out_ref):
    out_ref[...] = x_ref[...] + g_ref[...]


def _add_first(x_blk, g, n_tokens):
    blk_tokens, d = g.shape
    nblk = blk_tokens // _ROWBLK
    return pl.pallas_call(
        _add_first_body,
        grid=(nblk,),
        in_specs=[
            pl.BlockSpec((_ROWBLK, d), lambda i: (i, 0)),
            pl.BlockSpec((_ROWBLK, d), lambda i: (i, 0)),
        ],
        out_specs=pl.BlockSpec((_ROWBLK, d), lambda i: (i, 0)),
        out_shape=jax.ShapeDtypeStruct((n_tokens, d), jnp.float32),
    )(x_blk, g)


def _add_next_body(buf_ref, x_ref, g_ref, out_ref):
    del buf_ref
    out_ref[...] = x_ref[...] + g_ref[...]


def _add_next(buf, x_blk, g, k):
    n_tokens, d = buf.shape
    blk_tokens = g.shape[0]
    nblk = blk_tokens // _ROWBLK
    row0 = k * nblk
    return pl.pallas_call(
        _add_next_body,
        grid=(nblk,),
        in_specs=[
            pl.BlockSpec((8, d), lambda i: (0, 0)),
            pl.BlockSpec((_ROWBLK, d), lambda i: (i, 0)),
            pl.BlockSpec((_ROWBLK, d), lambda i: (i, 0)),
        ],
        out_specs=pl.BlockSpec((_ROWBLK, d), lambda i, r0=row0: (r0 + i, 0)),
        out_shape=jax.ShapeDtypeStruct((n_tokens, d), jnp.float32),
        input_output_aliases={0: 0},
    )(buf, x_blk, g)


@jax.jit
def kernel(x, coords, pos_embed):
    b, n, d = x.shape
    n_tokens = b * n
    x2d = x.reshape(n_tokens, d)
    ci32 = coords.astype(jnp.int32)
    c0 = ci32[..., 0].reshape(n_tokens)
    c1 = ci32[..., 1].reshape(n_tokens)
    table = pos_embed[0]

    blk = n_tokens // _KSPLIT
    gather = _make_gather(blk, d)
    gs = [
        gather(c0[k * blk:(k + 1) * blk], c1[k * blk:(k + 1) * blk], table)
        for k in range(_KSPLIT)
    ]
    buf = _add_first(x2d[0:blk], gs[0], n_tokens)
    for k in range(1, _KSPLIT):
        buf = _add_next(buf, x2d[k * blk:(k + 1) * blk], gs[k], k)
    return buf.reshape(b, n, d)


# speedup vs baseline: 1.3672x; 1.3215x over previous
import functools

import jax
import jax.numpy as jnp
from jax import lax
from jax.experimental import pallas as pl
from jax.experimental.pallas import tpu as pltpu
from jax.experimental.pallas import tpu_sc as plsc

_EMBED_DIM = 768
_NGRIDS = 256
_LANES = 16

_NC = 2
_NS = 16
_NW = _NC * _NS

_KSPLIT = 4
_CHUNK = 32
_NBUF = 4
_ROWBLK = 512


def _div224(v):
    return ((v >> 5) * 9363) >> 16


def _gather_body(blk_tokens, blk0, c0_hbm, c1_hbm, table_hbm, out_hbm,
                 c0_v, c1_v, idx_v, gb, *sems):
    rows_per_w = blk_tokens // _NW
    n_chunks = rows_per_w // _CHUNK
    gsem = sems[0:_NBUF]
    osem = sems[_NBUF:2 * _NBUF]
    wid = lax.axis_index("s") * _NC + lax.axis_index("c")
    base = wid * rows_per_w

    pltpu.sync_copy(c0_hbm.at[pl.ds(blk0 + base, rows_per_w)], c0_v)
    pltpu.sync_copy(c1_hbm.at[pl.ds(blk0 + base, rows_per_w)], c1_v)

    def idx_body(i, _):
        ci = i // (_CHUNK // _LANES)
        off = (i % (_CHUNK // _LANES)) * _LANES
        a = c0_v[pl.ds(i * _LANES, _LANES)]
        b = c1_v[pl.ds(i * _LANES, _LANES)]
        idx_v[ci, pl.ds(off, _LANES)] = _div224(a) * _NGRIDS + _div224(b)
        return 0

    lax.fori_loop(0, rows_per_w // _LANES, idx_body, 0)

    def out_slice(ci):
        return out_hbm.at[pl.ds(base + ci * _CHUNK, _CHUNK)]

    def stage_in(i, b):
        @pl.when(i >= _NBUF)
        def _():
            pltpu.make_async_copy(gb.at[b], out_slice(i), osem[b]).wait()
        pltpu.async_copy(table_hbm.at[idx_v.at[i]], gb.at[b], gsem[b])

    def stage_out(cj, bj):
        pltpu.make_async_copy(table_hbm.at[idx_v.at[cj]], gb.at[bj],
                              gsem[bj]).wait()
        pltpu.async_copy(gb.at[bj], out_slice(cj), osem[bj])

    def group_body(g, _):
        for b in range(_NBUF):
            i = g * _NBUF + b
            stage_in(i, b)

            @pl.when(i >= 1)
            def _():
                stage_out(i - 1, (b - 1) % _NBUF)
        return 0

    lax.fori_loop(0, n_chunks // _NBUF, group_body, 0)

    last = n_chunks - 1
    stage_out(last, last % _NBUF)
    for k in range(_NBUF):
        ci = n_chunks - _NBUF + k
        pltpu.make_async_copy(gb.at[ci % _NBUF], out_slice(ci),
                              osem[ci % _NBUF]).wait()


def _make_gather(blk_tokens, blk0, d):
    mesh = plsc.VectorSubcoreMesh(core_axis_name="c", subcore_axis_name="s")
    rows_per_w = blk_tokens // _NW
    return pl.kernel(
        functools.partial(_gather_body, blk_tokens, blk0),
        out_type=jax.ShapeDtypeStruct((blk_tokens, d), jnp.float32),
        mesh=mesh,
        scratch_types=[
            pltpu.VMEM((rows_per_w,), jnp.int32),
            pltpu.VMEM((rows_per_w,), jnp.int32),
            pltpu.VMEM((rows_per_w // _CHUNK, _CHUNK), jnp.int32),
            pltpu.VMEM((_NBUF, _CHUNK, d), jnp.float32),
        ] + [pltpu.SemaphoreType.DMA] * (2 * _NBUF),
    )


def _add_first_body(x_ref, g_ref, out_ref):
    out_ref[...] = x_ref[...] + g_ref[...]


def _add_first(x2d, g, n_tokens):
    blk_tokens, d = g.shape
    nblk = blk_tokens // _ROWBLK
    return pl.pallas_call(
        _add_first_body,
        grid=(nblk,),
        in_specs=[
            pl.BlockSpec((_ROWBLK, d), lambda i: (i, 0)),
            pl.BlockSpec((_ROWBLK, d), lambda i: (i, 0)),
        ],
        out_specs=pl.BlockSpec((_ROWBLK, d), lambda i: (i, 0)),
        out_shape=jax.ShapeDtypeStruct((n_tokens, d), jnp.float32),
    )(x2d, g)


def _add_next_body(buf_ref, x_ref, g_ref, out_ref):
    del buf_ref
    out_ref[...] = x_ref[...] + g_ref[...]


def _add_next(buf, x2d, g, k):
    n_tokens, d = buf.shape
    blk_tokens = g.shape[0]
    nblk = blk_tokens // _ROWBLK
    row0 = k * nblk
    return pl.pallas_call(
        _add_next_body,
        grid=(nblk,),
        in_specs=[
            pl.BlockSpec((8, d), lambda i: (0, 0)),
            pl.BlockSpec((_ROWBLK, d), lambda i, r0=row0: (r0 + i, 0)),
            pl.BlockSpec((_ROWBLK, d), lambda i: (i, 0)),
        ],
        out_specs=pl.BlockSpec((_ROWBLK, d), lambda i, r0=row0: (r0 + i, 0)),
        out_shape=jax.ShapeDtypeStruct((n_tokens, d), jnp.float32),
        input_output_aliases={0: 0},
    )(buf, x2d, g)


@jax.jit
def kernel(x, coords, pos_embed):
    b, n, d = x.shape
    n_tokens = b * n
    x2d = x.reshape(n_tokens, d)
    ci32 = coords.astype(jnp.int32)
    c0 = ci32[..., 0].reshape(n_tokens)
    c1 = ci32[..., 1].reshape(n_tokens)
    table = pos_embed[0]

    blk = n_tokens // _KSPLIT
    gs = [
        _make_gather(blk, k * blk, d)(c0, c1, table)
        for k in range(_KSPLIT)
    ]
    buf = _add_first(x2d, gs[0], n_tokens)
    for k in range(1, _KSPLIT):
        buf = _add_next(buf, x2d, gs[k], k)
    return buf.reshape(b, n, d)


# speedup vs baseline: 1.4872x; 1.0878x over previous
import functools

import jax
import jax.numpy as jnp
from jax import lax
from jax.experimental import pallas as pl
from jax.experimental.pallas import tpu as pltpu
from jax.experimental.pallas import tpu_sc as plsc

_EMBED_DIM = 768
_NGRIDS = 256
_LANES = 16

_NC = 2
_NS = 16
_NW = _NC * _NS

_CHUNK = 32
_NBUF = 2


def _div224(v):
    return ((v >> 5) * 9363) >> 16


def _slide_pe_body(n_tokens, x_hbm, c0_hbm, c1_hbm, table_hbm, out_hbm,
                   c0_v, c1_v, idx_v, xb, gb, *sems):
    rows_per_w = n_tokens // _NW
    n_chunks = rows_per_w // _CHUNK
    xsem = sems[0:_NBUF]
    gsem = sems[_NBUF:2 * _NBUF]
    osem = sems[2 * _NBUF:3 * _NBUF]
    wid = lax.axis_index("s") * _NC + lax.axis_index("c")
    base = wid * rows_per_w

    pltpu.sync_copy(c0_hbm.at[pl.ds(base, rows_per_w)], c0_v)
    pltpu.sync_copy(c1_hbm.at[pl.ds(base, rows_per_w)], c1_v)

    def idx_body(i, _):
        ci = i // (_CHUNK // _LANES)
        off = (i % (_CHUNK // _LANES)) * _LANES
        a = c0_v[pl.ds(i * _LANES, _LANES)]
        b = c1_v[pl.ds(i * _LANES, _LANES)]
        idx_v[ci, pl.ds(off, _LANES)] = _div224(a) * _NGRIDS + _div224(b)
        return 0

    lax.fori_loop(0, rows_per_w // _LANES, idx_body, 0)

    def x_slice(ci):
        return x_hbm.at[pl.ds(base + ci * _CHUNK, _CHUNK)]

    def out_slice(ci):
        return out_hbm.at[pl.ds(base + ci * _CHUNK, _CHUNK)]

    def stage_in(i, b):
        @pl.when(i >= _NBUF)
        def _():
            pltpu.make_async_copy(xb.at[b], out_slice(i), osem[b]).wait()
        pltpu.async_copy(x_slice(i), xb.at[b], xsem[b])
        pltpu.async_copy(table_hbm.at[idx_v.at[i]], gb.at[b], gsem[b])

    def stage_add_out(cj, bj):
        pltpu.make_async_copy(x_slice(cj), xb.at[bj], xsem[bj]).wait()
        pltpu.make_async_copy(table_hbm.at[idx_v.at[cj]], gb.at[bj],
                              gsem[bj]).wait()

        def add_body(r, _):
            for j in range(_EMBED_DIM // _LANES):
                sl = pl.ds(j * _LANES, _LANES)
                plsc.addupdate(xb.at[bj, r, sl], gb[bj, r, sl])
            return 0

        lax.fori_loop(0, _CHUNK, add_body, 0)
        pltpu.async_copy(xb.at[bj], out_slice(cj), osem[bj])

    def group_body(g, _):
        for b in range(_NBUF):
            i = g * _NBUF + b
            stage_in(i, b)

            @pl.when(i >= 1)
            def _():
                stage_add_out(i - 1, (b - 1) % _NBUF)
        return 0

    lax.fori_loop(0, n_chunks // _NBUF, group_body, 0)

    last = n_chunks - 1
    stage_add_out(last, last % _NBUF)
    for k in range(_NBUF):
        ci = n_chunks - _NBUF + k
        pltpu.make_async_copy(xb.at[ci % _NBUF], out_slice(ci),
                              osem[ci % _NBUF]).wait()


@jax.jit
def kernel(x, coords, pos_embed):
    b, n, d = x.shape
    n_tokens = b * n
    x2d = x.reshape(n_tokens, d)
    ci32 = coords.astype(jnp.int32)
    c0 = ci32[..., 0].reshape(n_tokens)
    c1 = ci32[..., 1].reshape(n_tokens)
    table = pos_embed[0]

    mesh = plsc.VectorSubcoreMesh(core_axis_name="c", subcore_axis_name="s")
    rows_per_w = n_tokens // _NW
    run = pl.kernel(
        functools.partial(_slide_pe_body, n_tokens),
        out_type=jax.ShapeDtypeStruct((n_tokens, d), jnp.float32),
        mesh=mesh,
        scratch_types=[
            pltpu.VMEM((rows_per_w,), jnp.int32),
            pltpu.VMEM((rows_per_w,), jnp.int32),
            pltpu.VMEM((rows_per_w // _CHUNK, _CHUNK), jnp.int32),
            pltpu.VMEM((_NBUF, _CHUNK, d), jnp.float32),
            pltpu.VMEM((_NBUF, _CHUNK, d), jnp.float32),
        ] + [pltpu.SemaphoreType.DMA] * (3 * _NBUF),
    )
    out = run(x2d, c0, c1, table)
    return out.reshape(b, n, d)
